# baseline (speedup 1.0000x reference)
import functools

import jax
import jax.numpy as jnp
from jax import lax
from jax.experimental import pallas as pl
from jax.experimental.pallas import tpu as pltpu
from jax.experimental.pallas import tpu_sc as plsc

N, DEG, D = 10000, 16, 256
L = 16
CHUNKS = D // L
NB = 8
NUM_BATCHES = N // NB
NW = 32

_mesh = plsc.VectorSubcoreMesh(core_axis_name="c", subcore_axis_name="s")


@functools.partial(
    pl.kernel,
    out_type=jax.ShapeDtypeStruct((N, 2 * D), jnp.float32),
    mesh=_mesh,
    scratch_types=[
        pltpu.VMEM((NB, DEG, D), jnp.float32),
        pltpu.VMEM((NB, 2 * D), jnp.float32),
    ],
)
def _mean_max(x_hbm, out_hbm, xv, ov):
    wid = lax.axis_index("s") * 2 + lax.axis_index("c")
    nbat = (NUM_BATCHES - wid + NW - 1) // NW

    def batch_body(t, _):
        base = (wid + t * NW) * NB
        pltpu.sync_copy(x_hbm.at[pl.ds(base, NB)], xv)

        def node_body(n, _):
            def chunk_body(j, _):
                col = j * L
                s = xv[n, 0, pl.ds(col, L)]
                m = s
                for i in range(1, DEG):
                    v = xv[n, i, pl.ds(col, L)]
                    s = s + v
                    m = jnp.maximum(m, v)
                ov[n, pl.ds(col, L)] = s * (1.0 / DEG)
                ov[n, pl.ds(D + col, L)] = m
                return 0

            return lax.fori_loop(0, CHUNKS, chunk_body, 0)

        lax.fori_loop(0, NB, node_body, 0)
        pltpu.sync_copy(ov, out_hbm.at[pl.ds(base, NB)])
        return 0

    lax.fori_loop(0, nbat, batch_body, 0)


def kernel(feat_dist):
    return _mean_max(feat_dist)


# speedup vs baseline: 1.4472x; 1.4472x over previous
import functools

import jax
import jax.numpy as jnp
from jax import lax
from jax.experimental import pallas as pl
from jax.experimental.pallas import tpu as pltpu
from jax.experimental.pallas import tpu_sc as plsc

N, DEG, D = 10000, 16, 256
L = 16
CHUNKS = D // L
NB = 8
NUM_BATCHES = N // NB
NW = 32

_mesh = plsc.VectorSubcoreMesh(core_axis_name="c", subcore_axis_name="s")


@functools.partial(
    pl.kernel,
    out_type=jax.ShapeDtypeStruct((N, 2 * D), jnp.float32),
    mesh=_mesh,
    scratch_types=[
        pltpu.VMEM((NB, DEG, D), jnp.float32),
        pltpu.VMEM((NB, 2 * D), jnp.float32),
    ],
)
def _mean_max(x_hbm, out_hbm, xv, ov):
    wid = lax.axis_index("s") * 2 + lax.axis_index("c")
    nbat = (NUM_BATCHES - wid + NW - 1) // NW

    def batch_body(t, _):
        base = (wid + t * NW) * NB
        pltpu.sync_copy(x_hbm.at[pl.ds(base, NB)], xv)

        @plsc.parallel_loop(0, NB * CHUNKS, 1, unroll=2)
        def _(it):
            n = it // CHUNKS
            col = (it % CHUNKS) * L
            vs = [xv[n, i, pl.ds(col, L)] for i in range(DEG)]
            ms = vs
            while len(vs) > 1:
                vs = [vs[k] + vs[k + 1] for k in range(0, len(vs), 2)]
                ms = [jnp.maximum(ms[k], ms[k + 1]) for k in range(0, len(ms), 2)]
            ov[n, pl.ds(col, L)] = vs[0] * (1.0 / DEG)
            ov[n, pl.ds(D + col, L)] = ms[0]
        pltpu.sync_copy(ov, out_hbm.at[pl.ds(base, NB)])
        return 0

    lax.fori_loop(0, nbat, batch_body, 0)


def kernel(feat_dist):
    return _mean_max(feat_dist)


# speedup vs baseline: 2.4990x; 1.7267x over previous
import jax
import jax.numpy as jnp
from jax import lax
from jax.experimental import pallas as pl
from jax.experimental.pallas import tpu as pltpu
from jax.experimental.pallas import tpu_sc as plsc

N, DEG, D = 10000, 16, 256
L = 16
CHUNKS = D // L
NB = 8
NUM_BATCHES = N // NB
NW = 32

_mesh = plsc.VectorSubcoreMesh(core_axis_name="c", subcore_axis_name="s")


def _tree_reduce_chunk(xslot, ovslot, it):
    n = it // CHUNKS
    col = (it % CHUNKS) * L
    vs = [xslot[n, i, pl.ds(col, L)] for i in range(DEG)]
    ms = vs
    while len(vs) > 1:
        vs = [vs[k] + vs[k + 1] for k in range(0, len(vs), 2)]
        ms = [jnp.maximum(ms[k], ms[k + 1]) for k in range(0, len(ms), 2)]
    ovslot[n, pl.ds(col, L)] = vs[0] * (1.0 / DEG)
    ovslot[n, pl.ds(D + col, L)] = ms[0]


def _kernel_body(x_hbm, out_hbm, xv0, xv1, ov0, ov1, is0, is1, os0, os1):
    wid = lax.axis_index("s") * 2 + lax.axis_index("c")
    nbat = (NUM_BATCHES - wid + NW - 1) // NW
    npairs = nbat // 2

    def base(t):
        return (wid + t * NW) * NB

    def start_in(t, xslot, sem):
        pltpu.async_copy(x_hbm.at[pl.ds(base(t), NB)], xslot, sem)

    def wait_in(t, xslot, sem):
        pltpu.make_async_copy(x_hbm.at[pl.ds(base(t), NB)], xslot, sem).wait()

    def start_out(t, ovslot, sem):
        pltpu.async_copy(ovslot, out_hbm.at[pl.ds(base(t), NB)], sem)

    def wait_out(t, ovslot, sem):
        pltpu.make_async_copy(ovslot, out_hbm.at[pl.ds(base(t), NB)], sem).wait()

    start_in(0, xv0, is0)
    start_in(1, xv1, is1)

    def pair_body(p, _):
        for k, (xs, ovs, isem, osem) in enumerate(
            ((xv0, ov0, is0, os0), (xv1, ov1, is1, os1))
        ):
            t = 2 * p + k
            wait_in(t, xs, isem)

            @pl.when(p > 0)
            def _():
                wait_out(t - 2, ovs, osem)

            @plsc.parallel_loop(0, NB * CHUNKS, 1, unroll=2)
            def _(it):
                _tree_reduce_chunk(xs, ovs, it)

            @pl.when(t + 2 < nbat)
            def _():
                start_in(t + 2, xs, isem)

            start_out(t, ovs, osem)
        return 0

    lax.fori_loop(0, npairs, pair_body, 0)

    @pl.when(nbat % 2 == 1)
    def _():
        t = nbat - 1
        wait_in(t, xv0, is0)
        wait_out(t - 2, ov0, os0)

        @plsc.parallel_loop(0, NB * CHUNKS, 1, unroll=2)
        def _(it):
            _tree_reduce_chunk(xv0, ov0, it)

        start_out(t, ov0, os0)
        wait_out(t, ov0, os0)
        wait_out(nbat - 2, ov1, os1)

    @pl.when(nbat % 2 == 0)
    def _():
        wait_out(nbat - 2, ov0, os0)
        wait_out(nbat - 1, ov1, os1)


_mean_max = pl.kernel(
    _kernel_body,
    out_type=jax.ShapeDtypeStruct((N, 2 * D), jnp.float32),
    mesh=_mesh,
    scratch_types=[
        pltpu.VMEM((NB, DEG, D), jnp.float32),
        pltpu.VMEM((NB, DEG, D), jnp.float32),
        pltpu.VMEM((NB, 2 * D), jnp.float32),
        pltpu.VMEM((NB, 2 * D), jnp.float32),
        pltpu.SemaphoreType.DMA,
        pltpu.SemaphoreType.DMA,
        pltpu.SemaphoreType.DMA,
        pltpu.SemaphoreType.DMA,
    ],
)


def kernel(feat_dist):
    return _mean_max(feat_dist)
